# Initial kernel scaffold; baseline (speedup 1.0000x reference)
import jax
import jax.numpy as jnp
from jax.experimental import pallas as pl


def kernel(x, embedding_weight):
    raise NotImplementedError("write your pallas kernel here")


# speedup vs baseline: 5.1070x; 5.1070x over previous
import functools

import numpy as np
import jax
import jax.numpy as jnp
from jax import lax
from jax.experimental import pallas as pl
from jax.experimental.pallas import tpu as pltpu
from jax.experimental.pallas import tpu_sc as plsc

MAX_LEN = 200
EMB_DIM = 32
N_VOCAB = 27
BATCH = 4096

NC, NS = 2, 16
NW = NC * NS
ROWS_TOTAL = BATCH * MAX_LEN
ROWS_PER_W = ROWS_TOTAL // NW
CHUNK = 800
NCHUNK = ROWS_PER_W // CHUNK
GSIZE = 80
NG = CHUNK // GSIZE


def _pe_table() -> np.ndarray:
    pos = np.expand_dims(np.arange(MAX_LEN), 1)
    pe = pos / np.power(
        10000, 2 * np.expand_dims(np.arange(EMB_DIM) // 2, 0) / EMB_DIM)
    pe[:, 0::2] = np.sin(pe[:, 0::2])
    pe[:, 1::2] = np.cos(pe[:, 1::2])
    return pe.astype(np.float32)


def _fuse_body(emb_ref, pe_ref, o_ref):
    o_ref[...] = emb_ref[...][None, :, :] + pe_ref[...][:, None, :]


def _fused_table(embedding_weight, pe):
    t3 = pl.pallas_call(
        _fuse_body,
        out_shape=jax.ShapeDtypeStruct((MAX_LEN, N_VOCAB, EMB_DIM), jnp.float32),
    )(embedding_weight, pe)
    return t3.reshape(MAX_LEN * N_VOCAB, EMB_DIM)


def _sc_body(table_hbm, x_hbm, loff_hbm, out_hbm, loff_v, x_v, idx_v, rows_v,
             gsem):
    wid = lax.axis_index("s") * NC + lax.axis_index("c")
    base = wid * ROWS_PER_W
    pltpu.sync_copy(loff_hbm, loff_v)

    def chunk_body(c, carry):
        cb = base + c * CHUNK
        pltpu.sync_copy(x_hbm.at[pl.ds(cb, CHUNK)], x_v)
        for i in range(CHUNK // 16):
            sl = pl.ds(i * 16, 16)
            idx_v[sl] = x_v[sl] + loff_v[sl]
        copies = []
        for g in range(NG):
            cp = pltpu.make_async_copy(
                table_hbm.at[idx_v.at[pl.ds(g * GSIZE, GSIZE)]],
                rows_v.at[pl.ds(g * GSIZE, GSIZE)],
                gsem,
            )
            cp.start()
            copies.append(cp)
        for cp in copies:
            cp.wait()
        pltpu.sync_copy(rows_v, out_hbm.at[pl.ds(cb, CHUNK)])
        return carry

    lax.fori_loop(0, NCHUNK, chunk_body, 0)


@functools.cache
def _make_sc_gather():
    return pl.kernel(
        _sc_body,
        out_type=jax.ShapeDtypeStruct((ROWS_TOTAL, EMB_DIM), jnp.float32),
        mesh=plsc.VectorSubcoreMesh(
            core_axis_name="c", subcore_axis_name="s", num_cores=NC,
            num_subcores=NS),
        scratch_types=[
            pltpu.VMEM((CHUNK,), jnp.int32),
            pltpu.VMEM((CHUNK,), jnp.int32),
            pltpu.VMEM((CHUNK,), jnp.int32),
            pltpu.VMEM((CHUNK, EMB_DIM), jnp.float32),
            pltpu.SemaphoreType.DMA,
        ],
        compiler_params=pltpu.CompilerParams(use_tc_tiling_on_sc=False),
    )


def kernel(x, embedding_weight):
    pe = jnp.asarray(_pe_table())
    table = _fused_table(embedding_weight, pe)
    x_flat = x.reshape(ROWS_TOTAL).astype(jnp.int32)
    loff = jnp.asarray(
        np.tile(N_VOCAB * np.arange(MAX_LEN, dtype=np.int32),
                CHUNK // MAX_LEN))
    out = _make_sc_gather()(table, x_flat, loff)
    return out.reshape(BATCH, MAX_LEN, EMB_DIM)


# speedup vs baseline: 5.3108x; 1.0399x over previous
import functools

import numpy as np
import jax
import jax.numpy as jnp
from jax import lax
from jax.experimental import pallas as pl
from jax.experimental.pallas import tpu as pltpu
from jax.experimental.pallas import tpu_sc as plsc

MAX_LEN = 200
EMB_DIM = 32
N_VOCAB = 27
BATCH = 4096

NC, NS = 2, 16
NW = NC * NS
ROWS_TOTAL = BATCH * MAX_LEN
ROWS_PER_W = ROWS_TOTAL // NW
LOFF = 800
GSIZE = 128
CHUNK = 1280
NG = CHUNK // GSIZE
NCHUNK = ROWS_PER_W // CHUNK


def _pe_table() -> np.ndarray:
    pos = np.expand_dims(np.arange(MAX_LEN), 1)
    pe = pos / np.power(
        10000, 2 * np.expand_dims(np.arange(EMB_DIM) // 2, 0) / EMB_DIM)
    pe[:, 0::2] = np.sin(pe[:, 0::2])
    pe[:, 1::2] = np.cos(pe[:, 1::2])
    return pe.astype(np.float32)


def _fuse_body(emb_ref, pe_ref, o_ref):
    o_ref[...] = emb_ref[...][None, :, :] + pe_ref[...][:, None, :]


def _fused_table(embedding_weight, pe):
    t3 = pl.pallas_call(
        _fuse_body,
        out_shape=jax.ShapeDtypeStruct((MAX_LEN, N_VOCAB, EMB_DIM), jnp.float32),
    )(embedding_weight, pe)
    return t3.reshape(MAX_LEN * N_VOCAB, EMB_DIM)


def _sc_body(table_hbm, x_hbm, loff_hbm, out_hbm, loff_v, idx_v,
             rows_v0, rows_v1, gsem0, gsem1, osem0, osem1):
    rows_v = (rows_v0, rows_v1)
    gsem = (gsem0, gsem1)
    osem = (osem0, osem1)
    wid = lax.axis_index("s") * NC + lax.axis_index("c")
    base = wid * ROWS_PER_W

    pltpu.sync_copy(loff_hbm, loff_v)
    pltpu.sync_copy(x_hbm.at[pl.ds(base, ROWS_PER_W)], idx_v)

    def idx_body(i, carry):
        sl = pl.ds(i * 16, 16)
        lsl = pl.ds((i % (LOFF // 16)) * 16, 16)
        idx_v[sl] = idx_v[sl] + loff_v[lsl]
        return carry

    lax.fori_loop(0, ROWS_PER_W // 16, idx_body, 0)

    def start_gathers(c):
        s = c % 2
        for g in range(NG):
            pltpu.make_async_copy(
                table_hbm.at[idx_v.at[pl.ds(c * CHUNK + g * GSIZE, GSIZE)]],
                rows_v[s].at[pl.ds(g * GSIZE, GSIZE)],
                gsem[s]).start()

    def drain_gathers(c):
        s = c % 2
        for g in range(NG):
            pltpu.make_async_copy(
                table_hbm.at[idx_v.at[pl.ds(c * CHUNK + g * GSIZE, GSIZE)]],
                rows_v[s].at[pl.ds(g * GSIZE, GSIZE)],
                gsem[s]).wait()

    def out_copy(c):
        s = c % 2
        return pltpu.make_async_copy(
            rows_v[s], out_hbm.at[pl.ds(base + c * CHUNK, CHUNK)], osem[s])

    start_gathers(0)
    for c in range(1, NCHUNK):
        if c >= 2:
            out_copy(c - 2).wait()
        start_gathers(c)
        drain_gathers(c - 1)
        out_copy(c - 1).start()
    out_copy(NCHUNK - 2).wait()
    drain_gathers(NCHUNK - 1)
    out_copy(NCHUNK - 1).start()
    out_copy(NCHUNK - 1).wait()


@functools.cache
def _make_sc_gather():
    return pl.kernel(
        _sc_body,
        out_type=jax.ShapeDtypeStruct((ROWS_TOTAL, EMB_DIM), jnp.float32),
        mesh=plsc.VectorSubcoreMesh(
            core_axis_name="c", subcore_axis_name="s", num_cores=NC,
            num_subcores=NS),
        scratch_types=[
            pltpu.VMEM((LOFF,), jnp.int32),
            pltpu.VMEM((ROWS_PER_W,), jnp.int32),
            pltpu.VMEM((CHUNK, EMB_DIM), jnp.float32),
            pltpu.VMEM((CHUNK, EMB_DIM), jnp.float32),
            pltpu.SemaphoreType.DMA,
            pltpu.SemaphoreType.DMA,
            pltpu.SemaphoreType.DMA,
            pltpu.SemaphoreType.DMA,
        ],
        compiler_params=pltpu.CompilerParams(use_tc_tiling_on_sc=False),
    )


def kernel(x, embedding_weight):
    pe = jnp.asarray(_pe_table())
    table = _fused_table(embedding_weight, pe)
    x_flat = x.reshape(ROWS_TOTAL).astype(jnp.int32)
    loff = jnp.asarray(
        np.tile(N_VOCAB * np.arange(MAX_LEN, dtype=np.int32),
                LOFF // MAX_LEN))
    out = _make_sc_gather()(table, x_flat, loff)
    return out.reshape(BATCH, MAX_LEN, EMB_DIM)
